# Initial kernel scaffold; baseline (speedup 1.0000x reference)
import functools

import jax
import jax.numpy as jnp
from jax import lax
from jax.experimental import pallas as pl
from jax.experimental.pallas import tpu as pltpu
from jax.experimental.pallas import tpu_sc as plsc

NUM_CORES = 2
NUM_SUBCORES = 16
NUM_WORKERS = NUM_CORES * NUM_SUBCORES

B_TOTAL = 16384 * 50
DIM = 32
B_PER_W = B_TOTAL // NUM_WORKERS
CHUNK = 3200
N_CHUNKS = B_PER_W // CHUNK


def _make_gather():
    mesh = plsc.VectorSubcoreMesh(core_axis_name="c", subcore_axis_name="s")

    @functools.partial(
        pl.kernel,
        out_type=jax.ShapeDtypeStruct((B_TOTAL, DIM), jnp.float32),
        mesh=mesh,
        scratch_types=[
            pltpu.VMEM((CHUNK,), jnp.int32),
            pltpu.VMEM((CHUNK, DIM), jnp.float32),
            pltpu.SemaphoreType.DMA,
        ],
    )
    def gather_kernel(idx_hbm, table_hbm, out_hbm, idx_v, rows_v, sem):
        wid = lax.axis_index("s") * NUM_CORES + lax.axis_index("c")
        base = wid * B_PER_W
        for c in range(N_CHUNKS):
            off = base + c * CHUNK
            pltpu.sync_copy(idx_hbm.at[pl.ds(off, CHUNK)], idx_v)
            pltpu.async_copy(table_hbm.at[idx_v], rows_v, sem).wait()
            pltpu.sync_copy(rows_v, out_hbm.at[pl.ds(off, CHUNK)])

    return gather_kernel


_gather = _make_gather()


@jax.jit
def kernel(token_ids, weight):
    flat_idx = token_ids.reshape(-1)
    out = _gather(flat_idx, weight)
    return out.reshape(token_ids.shape + (DIM,))


# speedup vs baseline: 1.1112x; 1.1112x over previous
import functools

import jax
import jax.numpy as jnp
from jax import lax
from jax.experimental import pallas as pl
from jax.experimental.pallas import tpu as pltpu
from jax.experimental.pallas import tpu_sc as plsc

NUM_CORES = 2
NUM_SUBCORES = 16
NUM_WORKERS = NUM_CORES * NUM_SUBCORES

B_TOTAL = 16384 * 50
DIM = 32
B_PER_W = B_TOTAL // NUM_WORKERS
CHUNK = 3200
N_CHUNKS = B_PER_W // CHUNK


def _make_gather():
    mesh = plsc.VectorSubcoreMesh(core_axis_name="c", subcore_axis_name="s")

    @functools.partial(
        pl.kernel,
        out_type=jax.ShapeDtypeStruct((B_TOTAL, DIM), jnp.float32),
        mesh=mesh,
        scratch_types=[
            pltpu.VMEM((CHUNK,), jnp.int32),
            pltpu.VMEM((CHUNK, DIM), jnp.float32),
            pltpu.SemaphoreType.DMA,
        ],
        compiler_params=pltpu.CompilerParams(use_tc_tiling_on_sc=False),
    )
    def gather_kernel(idx_hbm, table_hbm, out_hbm, idx_v, rows_v, sem):
        wid = lax.axis_index("s") * NUM_CORES + lax.axis_index("c")
        base = wid * B_PER_W
        for c in range(N_CHUNKS):
            off = base + c * CHUNK
            pltpu.sync_copy(idx_hbm.at[pl.ds(off, CHUNK)], idx_v)
            pltpu.async_copy(table_hbm.at[idx_v], rows_v, sem).wait()
            pltpu.sync_copy(rows_v, out_hbm.at[pl.ds(off, CHUNK)])

    return gather_kernel


_gather = _make_gather()


@jax.jit
def kernel(token_ids, weight):
    flat_idx = token_ids.reshape(-1)
    out = _gather(flat_idx, weight)
    return out.reshape(token_ids.shape + (DIM,))


# speedup vs baseline: 1.1124x; 1.0011x over previous
import functools

import jax
import jax.numpy as jnp
from jax import lax
from jax.experimental import pallas as pl
from jax.experimental.pallas import tpu as pltpu
from jax.experimental.pallas import tpu_sc as plsc

NUM_CORES = 2
NUM_SUBCORES = 16
NUM_WORKERS = NUM_CORES * NUM_SUBCORES

B_TOTAL = 16384 * 50
DIM = 32
B_PER_W = B_TOTAL // NUM_WORKERS
CHUNK = 1600
N_CHUNKS = B_PER_W // CHUNK


def _make_gather():
    mesh = plsc.VectorSubcoreMesh(core_axis_name="c", subcore_axis_name="s")

    @functools.partial(
        pl.kernel,
        out_type=jax.ShapeDtypeStruct((B_TOTAL, DIM), jnp.float32),
        mesh=mesh,
        scratch_types=[
            pltpu.VMEM((CHUNK,), jnp.int32),
            pltpu.VMEM((CHUNK,), jnp.int32),
            pltpu.VMEM((CHUNK, DIM), jnp.float32),
            pltpu.VMEM((CHUNK, DIM), jnp.float32),
            pltpu.SemaphoreType.DMA,
            pltpu.SemaphoreType.DMA,
            pltpu.SemaphoreType.DMA,
            pltpu.SemaphoreType.DMA,
        ],
        compiler_params=pltpu.CompilerParams(use_tc_tiling_on_sc=False),
    )
    def gather_kernel(idx_hbm, table_hbm, out_hbm,
                      idx0, idx1, rows0, rows1, gsem0, gsem1, ssem0, ssem1):
        wid = lax.axis_index("s") * NUM_CORES + lax.axis_index("c")
        base = wid * B_PER_W
        idx_v = (idx0, idx1)
        rows_v = (rows0, rows1)
        gsem = (gsem0, gsem1)
        ssem = (ssem0, ssem1)
        gathers = [None] * N_CHUNKS
        stores = [None] * N_CHUNKS
        pltpu.sync_copy(idx_hbm.at[pl.ds(base, CHUNK)], idx_v[0])
        gathers[0] = pltpu.async_copy(table_hbm.at[idx_v[0]], rows_v[0], gsem[0])
        for i in range(N_CHUNKS):
            b = i & 1
            nb = 1 - b
            if i + 1 < N_CHUNKS:
                off = base + (i + 1) * CHUNK
                pltpu.sync_copy(idx_hbm.at[pl.ds(off, CHUNK)], idx_v[nb])
                if i >= 1:
                    stores[i - 1].wait()
                gathers[i + 1] = pltpu.async_copy(
                    table_hbm.at[idx_v[nb]], rows_v[nb], gsem[nb])
            gathers[i].wait()
            stores[i] = pltpu.async_copy(
                rows_v[b], out_hbm.at[pl.ds(base + i * CHUNK, CHUNK)], ssem[b])
        stores[N_CHUNKS - 2].wait()
        stores[N_CHUNKS - 1].wait()

    return gather_kernel


_gather = _make_gather()


@jax.jit
def kernel(token_ids, weight):
    flat_idx = token_ids.reshape(-1)
    out = _gather(flat_idx, weight)
    return out.reshape(token_ids.shape + (DIM,))
